# baseline (speedup 1.0000x reference)
import jax
import jax.numpy as jnp
from jax import lax
from jax.experimental import pallas as pl
from jax.experimental.pallas import tpu as pltpu
from jax.experimental.pallas import tpu_sc as plsc

N = 10000
E = 320000
H = 128
C = 40

NC = 2
NS = 16
NW = NC * NS
CS = 128
NCHUNK = 80
EPWP = NCHUNK * CS
EPAD = NW * EPWP - E
G = 4
NWIN = NCHUNK // G
NSW = NWIN // 2
NP = 10240
RPT = NP // NS


def _sc_mesh():
    return plsc.VectorSubcoreMesh(core_axis_name="c", subcore_axis_name="s")


def _zero_stripe(zeros_hbm, buf_v, sh, row0):
    pltpu.sync_copy(zeros_hbm, buf_v)

    def piece(i, carry):
        pltpu.sync_copy(buf_v, sh.at[pl.ds(row0 + i * CS, CS)])
        return carry

    lax.fori_loop(0, RPT // CS, piece, None)


def _writeback_stripe(sh, buf_v, out_hbm, cid, row0):
    def piece(i, carry):
        pltpu.sync_copy(sh.at[pl.ds(row0 + i * CS, CS)], buf_v)
        pltpu.sync_copy(buf_v, out_hbm.at[cid, pl.ds(row0 + i * CS, CS)])
        return carry

    lax.fori_loop(0, RPT // CS, piece, None)


def _deg_body(dst_hbm, ones_hbm, zeros_hbm, out_hbm, deg_sh, dst_v, ones_v):
    cid = lax.axis_index("c")
    sid = lax.axis_index("s")
    wid = cid * NS + sid
    row0 = sid * RPT
    _zero_stripe(zeros_hbm, ones_v, deg_sh, row0)
    pltpu.sync_copy(ones_hbm, ones_v)
    pltpu.sync_copy(dst_hbm.at[wid], dst_v)
    plsc.subcore_barrier()

    def chunk(j, carry):
        pltpu.sync_copy(ones_v, deg_sh.at[dst_v.at[j]], add=True)
        return carry

    lax.fori_loop(0, NCHUNK, chunk, None)
    plsc.subcore_barrier()
    _writeback_stripe(deg_sh, ones_v, out_hbm, cid, row0)


def _sc_degree(dst_r, ones128, zeros128):
    f = pl.kernel(
        _deg_body,
        out_type=jax.ShapeDtypeStruct((NC, NP, H), jnp.float32),
        mesh=_sc_mesh(),
        scratch_types=[
            pltpu.VMEM_SHARED((NP, H), jnp.float32),
            pltpu.VMEM((NCHUNK, CS), jnp.int32),
            pltpu.VMEM((CS, H), jnp.float32),
        ],
    )
    return f(dst_r, ones128, zeros128)


def _acc_body(h_hbm, idx_hbm, zeros_hbm, out_hbm,
              acc_sh, slot0, slot1, rows0, rows1,
              semw0, semw1, semg0, semg1):
    cid = lax.axis_index("c")
    sid = lax.axis_index("s")
    wid = cid * NS + sid
    row0 = sid * RPT
    rows = (rows0, rows1)
    semg = (semg0, semg1)
    _zero_stripe(zeros_hbm, rows0, acc_sh, row0)
    plsc.subcore_barrier()

    pltpu.async_copy(idx_hbm.at[wid, 0], slot0, semw0).wait()
    pltpu.async_copy(idx_hbm.at[wid, 1], slot1, semw1)
    pltpu.async_copy(h_hbm.at[slot0.at[0, 0]], rows0, semg0)

    def chunk(slot, nslot, k, p):
        pltpu.make_async_copy(h_hbm.at[slot.at[0, k]], rows[p], semg[p]).wait()
        nk = 0 if k == G - 1 else k + 1
        nsl = nslot if k == G - 1 else slot
        if nsl is not None:
            pltpu.async_copy(h_hbm.at[nsl.at[0, nk]], rows[1 - p], semg[1 - p])
        pltpu.sync_copy(rows[p], acc_sh.at[slot.at[1, k]], add=True)

    def superwindow(w, carry):
        pltpu.make_async_copy(idx_hbm.at[wid, 0], slot1, semw1).wait()
        for k in range(G):
            chunk(slot0, slot1, k, k % 2)

        @pl.when(w < NSW - 1)
        def _fetch_even():
            pltpu.async_copy(idx_hbm.at[wid, 2 * w + 2], slot0, semw0)

        for k in range(G - 1):
            chunk(slot1, None, k, k % 2)

        @pl.when(w < NSW - 1)
        def _last_mid():
            pltpu.make_async_copy(idx_hbm.at[wid, 0], slot0, semw0).wait()
            chunk(slot1, slot0, G - 1, (G - 1) % 2)
            pltpu.async_copy(idx_hbm.at[wid, 2 * w + 3], slot1, semw1)

        @pl.when(w == NSW - 1)
        def _last_end():
            p = (G - 1) % 2
            pltpu.make_async_copy(h_hbm.at[slot1.at[0, G - 1]], rows[p],
                                  semg[p]).wait()
            pltpu.sync_copy(rows[p], acc_sh.at[slot1.at[1, G - 1]], add=True)

        return carry

    lax.fori_loop(0, NSW, superwindow, None)
    plsc.subcore_barrier()
    _writeback_stripe(acc_sh, rows0, out_hbm, cid, row0)


def _sc_accumulate(h_s, idx_packed, zeros128):
    f = pl.kernel(
        _acc_body,
        out_type=jax.ShapeDtypeStruct((NC, NP, H), jnp.float32),
        mesh=_sc_mesh(),
        scratch_types=[
            pltpu.VMEM_SHARED((NP, H), jnp.float32),
            pltpu.VMEM((2, G, CS), jnp.int32),
            pltpu.VMEM((2, G, CS), jnp.int32),
            pltpu.VMEM((CS, H), jnp.float32),
            pltpu.VMEM((CS, H), jnp.float32),
            pltpu.SemaphoreType.DMA,
            pltpu.SemaphoreType.DMA,
            pltpu.SemaphoreType.DMA,
            pltpu.SemaphoreType.DMA,
        ],
    )
    return f(h_s, idx_packed, zeros128)


R = 400


def _dinv(deg0, deg1):
    return lax.rsqrt(deg0[:, 0:1] + deg1[:, 0:1] + 1.0)


def _first_body(x_ref, w_ref, deg0_ref, deg1_ref, o_ref):
    dinv = _dinv(deg0_ref[...], deg1_ref[...])
    h = jnp.dot(x_ref[...], w_ref[...], preferred_element_type=jnp.float32)
    o_ref[...] = h * dinv


def _mid_body(a0_ref, a1_ref, hs_ref, deg0_ref, deg1_ref, b_ref, w_ref, o_ref):
    dinv = _dinv(deg0_ref[...], deg1_ref[...])
    z = a0_ref[...] + a1_ref[...] + hs_ref[...]
    z = jnp.maximum(z * dinv + b_ref[...], 0.0)
    o_ref[...] = jnp.dot(z, w_ref[...], preferred_element_type=jnp.float32) * dinv


def _head_body(a0_ref, a1_ref, hs_ref, deg0_ref, deg1_ref, b3_ref,
               wl1_ref, bl1_ref, wl2_ref, bl2_ref, o_ref):
    dinv = _dinv(deg0_ref[...], deg1_ref[...])
    z = a0_ref[...] + a1_ref[...] + hs_ref[...]
    z = jnp.maximum(z * dinv + b3_ref[...], 0.0)
    h4 = jnp.dot(z, wl1_ref[...], preferred_element_type=jnp.float32)
    h4 = jnp.maximum(h4 + bl1_ref[...], 0.0)
    logits = jnp.dot(h4, wl2_ref[...], preferred_element_type=jnp.float32)
    logits = logits + bl2_ref[...]
    col = lax.broadcasted_iota(jnp.int32, logits.shape, 1)
    logits = jnp.where(col < C, logits, -1e30)
    m = jnp.max(logits, axis=-1, keepdims=True)
    lse = jnp.log(jnp.sum(jnp.exp(logits - m), axis=-1, keepdims=True))
    o_ref[...] = logits - m - lse


def _row_spec(width):
    return pl.BlockSpec((R, width), lambda i: (i, 0))


def _full_spec(shape):
    return pl.BlockSpec(shape, lambda i: (0,) * len(shape))


def _tc_first(x, w, deg0, deg1):
    return pl.pallas_call(
        _first_body,
        grid=(N // R,),
        in_specs=[_row_spec(H), _full_spec((H, H)), _row_spec(16), _row_spec(16)],
        out_specs=_row_spec(H),
        out_shape=jax.ShapeDtypeStruct((N, H), jnp.float32),
    )(x, w, deg0, deg1)


def _tc_mid(a0, a1, hs, deg0, deg1, b, w):
    return pl.pallas_call(
        _mid_body,
        grid=(N // R,),
        in_specs=[_row_spec(H), _row_spec(H), _row_spec(H), _row_spec(16),
                  _row_spec(16), _full_spec((1, H)), _full_spec((H, H))],
        out_specs=_row_spec(H),
        out_shape=jax.ShapeDtypeStruct((N, H), jnp.float32),
    )(a0, a1, hs, deg0, deg1, b, w)


def _tc_head(a0, a1, hs, deg0, deg1, b3, wl1, bl1, wl2p, bl2p):
    return pl.pallas_call(
        _head_body,
        grid=(N // R,),
        in_specs=[_row_spec(H), _row_spec(H), _row_spec(H), _row_spec(16),
                  _row_spec(16), _full_spec((1, H)), _full_spec((H, H)),
                  _full_spec((1, H)), _full_spec((H, H)), _full_spec((1, H))],
        out_specs=_row_spec(H),
        out_shape=jax.ShapeDtypeStruct((N, H), jnp.float32),
    )(a0, a1, hs, deg0, deg1, b3, wl1, bl1, wl2p, bl2p)


def kernel(x, edge_index, batch, W1, b1, W2, b2, W3, b3, Wl1, bl1, Wl2, bl2):
    del batch
    srcp = jnp.concatenate([edge_index[0], jnp.zeros((EPAD,), jnp.int32)])
    dstp = jnp.concatenate([edge_index[1],
                            jnp.full((EPAD,), NP - 1, jnp.int32)])
    dst_r = dstp.reshape(NW, NCHUNK, CS)
    idx_packed = jnp.stack([srcp.reshape(NW, NWIN, G, CS),
                            dstp.reshape(NW, NWIN, G, CS)], axis=2)
    ones128 = jnp.ones((CS, H), jnp.float32)
    zeros128 = jnp.zeros((CS, H), jnp.float32)

    deg = _sc_degree(dst_r, ones128, zeros128)
    deg0, deg1 = deg[0, :N, :16], deg[1, :N, :16]

    hs = _tc_first(x, W1, deg0, deg1)
    acc = _sc_accumulate(hs, idx_packed, zeros128)
    hs = _tc_mid(acc[0, :N], acc[1, :N], hs, deg0, deg1, b1.reshape(1, H), W2)
    acc = _sc_accumulate(hs, idx_packed, zeros128)
    hs = _tc_mid(acc[0, :N], acc[1, :N], hs, deg0, deg1, b2.reshape(1, H), W3)
    acc = _sc_accumulate(hs, idx_packed, zeros128)

    wl2p = jnp.pad(Wl2, ((0, 0), (0, H - C)))
    bl2p = jnp.pad(bl2, (0, H - C)).reshape(1, H)
    out = _tc_head(acc[0, :N], acc[1, :N], hs, deg0, deg1, b3.reshape(1, H),
                   Wl1, bl1.reshape(1, H), wl2p, bl2p)
    return out[:, :C]


# speedup vs baseline: 2.7346x; 2.7346x over previous
import jax
import jax.numpy as jnp
from jax import lax
from jax.experimental import pallas as pl
from jax.experimental.pallas import tpu as pltpu
from jax.experimental.pallas import tpu_sc as plsc

N = 10000
E = 320000
H = 128
C = 40

NC = 2
NS = 16
NW = NC * NS
CS = 128
NCHUNK = 80
EPWP = NCHUNK * CS
EPAD = NW * EPWP - E
G = 4
NWIN = NCHUNK // G
NSW = NWIN // 2
NP = 10240
RPT = NP // NS


def _sc_mesh():
    return plsc.VectorSubcoreMesh(core_axis_name="c", subcore_axis_name="s")


def _zero_stripe(zeros_hbm, buf_v, sh, row0):
    pltpu.sync_copy(zeros_hbm, buf_v)

    def piece(i, carry):
        pltpu.sync_copy(buf_v, sh.at[pl.ds(row0 + i * CS, CS)])
        return carry

    lax.fori_loop(0, RPT // CS, piece, None)


def _writeback_stripe(sh, buf_v, out_hbm, cid, row0):
    def piece(i, carry):
        pltpu.sync_copy(sh.at[pl.ds(row0 + i * CS, CS)], buf_v)
        pltpu.sync_copy(buf_v, out_hbm.at[cid, pl.ds(row0 + i * CS, CS)])
        return carry

    lax.fori_loop(0, RPT // CS, piece, None)


def _deg_body(dst_hbm, ones_hbm, zeros_hbm, out_hbm, deg_sh, dst_v, ones_v):
    cid = lax.axis_index("c")
    sid = lax.axis_index("s")
    wid = cid * NS + sid
    row0 = sid * RPT
    _zero_stripe(zeros_hbm, ones_v, deg_sh, row0)
    pltpu.sync_copy(ones_hbm, ones_v)
    pltpu.sync_copy(dst_hbm.at[wid], dst_v)
    plsc.subcore_barrier()

    def chunk(j, carry):
        pltpu.sync_copy(ones_v, deg_sh.at[dst_v.at[j]], add=True)
        return carry

    lax.fori_loop(0, NCHUNK, chunk, None)
    plsc.subcore_barrier()
    _writeback_stripe(deg_sh, ones_v, out_hbm, cid, row0)


def _sc_degree(dst_r, ones128, zeros128):
    f = pl.kernel(
        _deg_body,
        out_type=jax.ShapeDtypeStruct((NC, NP, H), jnp.float32),
        mesh=_sc_mesh(),
        scratch_types=[
            pltpu.VMEM_SHARED((NP, H), jnp.float32),
            pltpu.VMEM((NCHUNK, CS), jnp.int32),
            pltpu.VMEM((CS, H), jnp.float32),
        ],
    )
    return f(dst_r, ones128, zeros128)


def _acc_body(h_hbm, idx_hbm, zeros_hbm, out_hbm,
              acc_sh, slot0, slot1, rows0, rows1,
              semw0, semw1, semg0, semg1):
    cid = lax.axis_index("c")
    sid = lax.axis_index("s")
    wid = cid * NS + sid
    row0 = sid * RPT
    rows = (rows0, rows1)
    semg = (semg0, semg1)
    _zero_stripe(zeros_hbm, rows0, acc_sh, row0)
    plsc.subcore_barrier()

    pltpu.async_copy(idx_hbm.at[wid, 0], slot0, semw0).wait()
    pltpu.async_copy(idx_hbm.at[wid, 1], slot1, semw1)
    pltpu.async_copy(h_hbm.at[slot0.at[0, 0]], rows0, semg0)

    def chunk(slot, nslot, k, p):
        pltpu.make_async_copy(h_hbm.at[slot.at[0, k]], rows[p], semg[p]).wait()
        nk = 0 if k == G - 1 else k + 1
        nsl = nslot if k == G - 1 else slot
        if nsl is not None:
            pltpu.async_copy(h_hbm.at[nsl.at[0, nk]], rows[1 - p], semg[1 - p])
        pltpu.sync_copy(rows[p], acc_sh.at[slot.at[1, k]], add=True)

    def superwindow(w, carry):
        pltpu.make_async_copy(idx_hbm.at[wid, 0], slot1, semw1).wait()
        for k in range(G):
            chunk(slot0, slot1, k, k % 2)

        @pl.when(w < NSW - 1)
        def _fetch_even():
            pltpu.async_copy(idx_hbm.at[wid, 2 * w + 2], slot0, semw0)

        for k in range(G - 1):
            chunk(slot1, None, k, k % 2)

        @pl.when(w < NSW - 1)
        def _last_mid():
            pltpu.make_async_copy(idx_hbm.at[wid, 0], slot0, semw0).wait()
            chunk(slot1, slot0, G - 1, (G - 1) % 2)
            pltpu.async_copy(idx_hbm.at[wid, 2 * w + 3], slot1, semw1)

        @pl.when(w == NSW - 1)
        def _last_end():
            p = (G - 1) % 2
            pltpu.make_async_copy(h_hbm.at[slot1.at[0, G - 1]], rows[p],
                                  semg[p]).wait()
            pltpu.sync_copy(rows[p], acc_sh.at[slot1.at[1, G - 1]], add=True)

        return carry

    lax.fori_loop(0, NSW, superwindow, None)
    plsc.subcore_barrier()
    _writeback_stripe(acc_sh, rows0, out_hbm, cid, row0)


def _sc_accumulate(h_s, idx_packed, zeros128):
    f = pl.kernel(
        _acc_body,
        out_type=jax.ShapeDtypeStruct((NC, NP, H), jnp.float32),
        mesh=_sc_mesh(),
        scratch_types=[
            pltpu.VMEM_SHARED((NP, H), jnp.float32),
            pltpu.VMEM((2, G, CS), jnp.int32),
            pltpu.VMEM((2, G, CS), jnp.int32),
            pltpu.VMEM((CS, H), jnp.float32),
            pltpu.VMEM((CS, H), jnp.float32),
            pltpu.SemaphoreType.DMA,
            pltpu.SemaphoreType.DMA,
            pltpu.SemaphoreType.DMA,
            pltpu.SemaphoreType.DMA,
        ],
    )
    return f(h_s, idx_packed, zeros128)


R = 400


def _dinv(deg0, deg1):
    return lax.rsqrt(deg0[:, 0:1] + deg1[:, 0:1] + 1.0)


def _first_body(x_ref, w_ref, deg0_ref, deg1_ref, o_ref):
    dinv = _dinv(deg0_ref[...], deg1_ref[...])
    h = jnp.dot(x_ref[...], w_ref[...], preferred_element_type=jnp.float32)
    o_ref[...] = h * dinv


def _mid_body(a0_ref, a1_ref, hs_ref, deg0_ref, deg1_ref, b_ref, w_ref, o_ref):
    dinv = _dinv(deg0_ref[...], deg1_ref[...])
    z = a0_ref[...] + a1_ref[...] + hs_ref[...]
    z = jnp.maximum(z * dinv + b_ref[...], 0.0)
    o_ref[...] = jnp.dot(z, w_ref[...], preferred_element_type=jnp.float32) * dinv


def _head_body(a0_ref, a1_ref, hs_ref, deg0_ref, deg1_ref, b3_ref,
               wl1_ref, bl1_ref, wl2_ref, bl2_ref, o_ref):
    dinv = _dinv(deg0_ref[...], deg1_ref[...])
    z = a0_ref[...] + a1_ref[...] + hs_ref[...]
    z = jnp.maximum(z * dinv + b3_ref[...], 0.0)
    h4 = jnp.dot(z, wl1_ref[...], preferred_element_type=jnp.float32)
    h4 = jnp.maximum(h4 + bl1_ref[...], 0.0)
    logits = jnp.dot(h4, wl2_ref[...], preferred_element_type=jnp.float32)
    logits = logits + bl2_ref[...]
    col = lax.broadcasted_iota(jnp.int32, logits.shape, 1)
    logits = jnp.where(col < C, logits, -1e30)
    m = jnp.max(logits, axis=-1, keepdims=True)
    lse = jnp.log(jnp.sum(jnp.exp(logits - m), axis=-1, keepdims=True))
    o_ref[...] = logits - m - lse


def _row_spec(width):
    return pl.BlockSpec((R, width), lambda i: (i, 0))


def _full_spec(shape):
    return pl.BlockSpec(shape, lambda i: (0,) * len(shape))


def _tc_first(x, w, deg0, deg1):
    return pl.pallas_call(
        _first_body,
        grid=(N // R,),
        in_specs=[_row_spec(H), _full_spec((H, H)), _row_spec(16), _row_spec(16)],
        out_specs=_row_spec(H),
        out_shape=jax.ShapeDtypeStruct((N, H), jnp.float32),
    )(x, w, deg0, deg1)


def _tc_mid(a0, a1, hs, deg0, deg1, b, w):
    return pl.pallas_call(
        _mid_body,
        grid=(N // R,),
        in_specs=[_row_spec(H), _row_spec(H), _row_spec(H), _row_spec(16),
                  _row_spec(16), _full_spec((1, H)), _full_spec((H, H))],
        out_specs=_row_spec(H),
        out_shape=jax.ShapeDtypeStruct((N, H), jnp.float32),
    )(a0, a1, hs, deg0, deg1, b, w)


def _tc_head(a0, a1, hs, deg0, deg1, b3, wl1, bl1, wl2p, bl2p):
    return pl.pallas_call(
        _head_body,
        grid=(N // R,),
        in_specs=[_row_spec(H), _row_spec(H), _row_spec(H), _row_spec(16),
                  _row_spec(16), _full_spec((1, H)), _full_spec((H, H)),
                  _full_spec((1, H)), _full_spec((H, H)), _full_spec((1, H))],
        out_specs=_row_spec(H),
        out_shape=jax.ShapeDtypeStruct((N, H), jnp.float32),
    )(a0, a1, hs, deg0, deg1, b3, wl1, bl1, wl2p, bl2p)


def kernel(x, edge_index, batch, W1, b1, W2, b2, W3, b3, Wl1, bl1, Wl2, bl2):
    del batch
    pad = jnp.arange(EPAD, dtype=jnp.int32)
    srcp = jnp.concatenate([edge_index[0], pad % N])
    dstp = jnp.concatenate([edge_index[1], N + pad % (NP - N)])
    dst_r = dstp.reshape(NW, NCHUNK, CS)
    idx_packed = jnp.stack([srcp.reshape(NW, NWIN, G, CS),
                            dstp.reshape(NW, NWIN, G, CS)], axis=2)
    ones128 = jnp.ones((CS, H), jnp.float32)
    zeros128 = jnp.zeros((CS, H), jnp.float32)

    deg = _sc_degree(dst_r, ones128, zeros128)
    deg0, deg1 = deg[0, :N, :16], deg[1, :N, :16]

    hs = _tc_first(x, W1, deg0, deg1)
    acc = _sc_accumulate(hs, idx_packed, zeros128)
    hs = _tc_mid(acc[0, :N], acc[1, :N], hs, deg0, deg1, b1.reshape(1, H), W2)
    acc = _sc_accumulate(hs, idx_packed, zeros128)
    hs = _tc_mid(acc[0, :N], acc[1, :N], hs, deg0, deg1, b2.reshape(1, H), W3)
    acc = _sc_accumulate(hs, idx_packed, zeros128)

    wl2p = jnp.pad(Wl2, ((0, 0), (0, H - C)))
    bl2p = jnp.pad(bl2, (0, H - C)).reshape(1, H)
    out = _tc_head(acc[0, :N], acc[1, :N], hs, deg0, deg1, b3.reshape(1, H),
                   Wl1, bl1.reshape(1, H), wl2p, bl2p)
    return out[:, :C]


# speedup vs baseline: 2.7924x; 1.0211x over previous
import jax
import jax.numpy as jnp
from jax import lax
from jax.experimental import pallas as pl
from jax.experimental.pallas import tpu as pltpu
from jax.experimental.pallas import tpu_sc as plsc

N = 10000
E = 320000
H = 128
C = 40

NC = 2
NS = 16
NW = NC * NS
CS = 128
NCHUNK = 80
EPWP = NCHUNK * CS
EPAD = NW * EPWP - E
G = 4
NWIN = NCHUNK // G
NSW = NWIN // 2
NP = 10240
RPT = NP // NS


def _sc_mesh():
    return plsc.VectorSubcoreMesh(core_axis_name="c", subcore_axis_name="s")


def _zero_stripe(zeros_hbm, buf_v, sh, row0):
    pltpu.sync_copy(zeros_hbm, buf_v)

    def piece(i, carry):
        pltpu.sync_copy(buf_v, sh.at[pl.ds(row0 + i * CS, CS)])
        return carry

    lax.fori_loop(0, RPT // CS, piece, None)


def _writeback_stripe(sh, buf_v, out_hbm, cid, row0):
    def piece(i, carry):
        pltpu.sync_copy(sh.at[pl.ds(row0 + i * CS, CS)], buf_v)
        pltpu.sync_copy(buf_v, out_hbm.at[cid, pl.ds(row0 + i * CS, CS)])
        return carry

    lax.fori_loop(0, RPT // CS, piece, None)


def _deg_body(dst_hbm, ones_hbm, zeros_hbm, out_hbm, deg_sh, dst_v, ones_v):
    cid = lax.axis_index("c")
    sid = lax.axis_index("s")
    wid = cid * NS + sid
    row0 = sid * RPT
    _zero_stripe(zeros_hbm, ones_v, deg_sh, row0)
    pltpu.sync_copy(ones_hbm, ones_v)
    pltpu.sync_copy(dst_hbm.at[wid], dst_v)
    plsc.subcore_barrier()

    def chunk(j, carry):
        pltpu.sync_copy(ones_v, deg_sh.at[dst_v.at[j]], add=True)
        return carry

    lax.fori_loop(0, NCHUNK, chunk, None)
    plsc.subcore_barrier()
    _writeback_stripe(deg_sh, ones_v, out_hbm, cid, row0)


def _sc_degree(dst_r, ones128, zeros128):
    f = pl.kernel(
        _deg_body,
        out_type=jax.ShapeDtypeStruct((NC, NP, H), jnp.float32),
        mesh=_sc_mesh(),
        scratch_types=[
            pltpu.VMEM_SHARED((NP, H), jnp.float32),
            pltpu.VMEM((NCHUNK, CS), jnp.int32),
            pltpu.VMEM((CS, H), jnp.float32),
        ],
    )
    return f(dst_r, ones128, zeros128)


def _acc_body(h_hbm, idx_hbm, zeros_hbm, out_hbm,
              acc_sh, slot0, slot1, rows0, rows1,
              semw0, semw1, semg0, semg1):
    cid = lax.axis_index("c")
    sid = lax.axis_index("s")
    wid = cid * NS + sid
    row0 = sid * RPT
    rows = (rows0, rows1)
    semg = (semg0, semg1)
    _zero_stripe(zeros_hbm, rows0, acc_sh, row0)
    plsc.subcore_barrier()

    pltpu.async_copy(idx_hbm.at[wid, 0], slot0, semw0).wait()
    pltpu.async_copy(idx_hbm.at[wid, 1], slot1, semw1)
    pltpu.async_copy(h_hbm.at[slot0.at[0, 0]], rows0, semg0)

    def chunk(slot, nslot, k, p):
        pltpu.make_async_copy(h_hbm.at[slot.at[0, k]], rows[p], semg[p]).wait()
        nk = 0 if k == G - 1 else k + 1
        nsl = nslot if k == G - 1 else slot
        if nsl is not None:
            pltpu.async_copy(h_hbm.at[nsl.at[0, nk]], rows[1 - p], semg[1 - p])
        pltpu.sync_copy(rows[p], acc_sh.at[slot.at[1, k]], add=True)

    def superwindow(w, carry):
        pltpu.make_async_copy(idx_hbm.at[wid, 0], slot1, semw1).wait()
        for k in range(G):
            chunk(slot0, slot1, k, k % 2)

        @pl.when(w < NSW - 1)
        def _fetch_even():
            pltpu.async_copy(idx_hbm.at[wid, 2 * w + 2], slot0, semw0)

        for k in range(G - 1):
            chunk(slot1, None, k, k % 2)

        @pl.when(w < NSW - 1)
        def _last_mid():
            pltpu.make_async_copy(idx_hbm.at[wid, 0], slot0, semw0).wait()
            chunk(slot1, slot0, G - 1, (G - 1) % 2)
            pltpu.async_copy(idx_hbm.at[wid, 2 * w + 3], slot1, semw1)

        @pl.when(w == NSW - 1)
        def _last_end():
            p = (G - 1) % 2
            pltpu.make_async_copy(h_hbm.at[slot1.at[0, G - 1]], rows[p],
                                  semg[p]).wait()
            pltpu.sync_copy(rows[p], acc_sh.at[slot1.at[1, G - 1]], add=True)

        return carry

    lax.fori_loop(0, NSW, superwindow, None)
    plsc.subcore_barrier()
    _writeback_stripe(acc_sh, rows0, out_hbm, cid, row0)


def _sc_accumulate(h_s, idx_packed, zeros128):
    f = pl.kernel(
        _acc_body,
        out_type=jax.ShapeDtypeStruct((NC, NP, H), jnp.float32),
        mesh=_sc_mesh(),
        scratch_types=[
            pltpu.VMEM_SHARED((NP, H), jnp.float32),
            pltpu.VMEM((2, G, CS), jnp.int32),
            pltpu.VMEM((2, G, CS), jnp.int32),
            pltpu.VMEM((CS, H), jnp.float32),
            pltpu.VMEM((CS, H), jnp.float32),
            pltpu.SemaphoreType.DMA,
            pltpu.SemaphoreType.DMA,
            pltpu.SemaphoreType.DMA,
            pltpu.SemaphoreType.DMA,
        ],
    )
    return f(h_s, idx_packed, zeros128)


R = 400


def _dinv(deg0, deg1):
    return lax.rsqrt(deg0[:, 0:1] + deg1[:, 0:1] + 1.0)


def _first_body(x_ref, w_ref, deg0_ref, deg1_ref, o_ref):
    dinv = _dinv(deg0_ref[...], deg1_ref[...])
    h = jnp.dot(x_ref[...], w_ref[...], preferred_element_type=jnp.float32)
    o_ref[...] = h * dinv


def _mid_body(a0_ref, a1_ref, hs_ref, deg0_ref, deg1_ref, b_ref, w_ref, o_ref):
    dinv = _dinv(deg0_ref[...], deg1_ref[...])
    z = a0_ref[0] + a1_ref[0] + hs_ref[...]
    z = jnp.maximum(z * dinv + b_ref[...], 0.0)
    o_ref[...] = jnp.dot(z, w_ref[...], preferred_element_type=jnp.float32) * dinv


def _head_body(a0_ref, a1_ref, hs_ref, deg0_ref, deg1_ref, b3_ref,
               wl1_ref, bl1_ref, wl2_ref, bl2_ref, o_ref):
    dinv = _dinv(deg0_ref[...], deg1_ref[...])
    z = a0_ref[0] + a1_ref[0] + hs_ref[...]
    z = jnp.maximum(z * dinv + b3_ref[...], 0.0)
    h4 = jnp.dot(z, wl1_ref[...], preferred_element_type=jnp.float32)
    h4 = jnp.maximum(h4 + bl1_ref[...], 0.0)
    logits = jnp.dot(h4, wl2_ref[...], preferred_element_type=jnp.float32)
    logits = logits + bl2_ref[...]
    col = lax.broadcasted_iota(jnp.int32, logits.shape, 1)
    logits = jnp.where(col < C, logits, -1e30)
    m = jnp.max(logits, axis=-1, keepdims=True)
    lse = jnp.log(jnp.sum(jnp.exp(logits - m), axis=-1, keepdims=True))
    o_ref[...] = logits - m - lse


def _row_spec(width):
    return pl.BlockSpec((R, width), lambda i: (i, 0))


def _acc_spec(core):
    return pl.BlockSpec((1, R, H), lambda i, core=core: (core, i, 0))


def _full_spec(shape):
    return pl.BlockSpec(shape, lambda i: (0,) * len(shape))


def _tc_first(x, w, deg0, deg1):
    return pl.pallas_call(
        _first_body,
        grid=(N // R,),
        in_specs=[_row_spec(H), _full_spec((H, H)), _row_spec(16), _row_spec(16)],
        out_specs=_row_spec(H),
        out_shape=jax.ShapeDtypeStruct((N, H), jnp.float32),
    )(x, w, deg0, deg1)


def _tc_mid(acc, hs, deg0, deg1, b, w):
    return pl.pallas_call(
        _mid_body,
        grid=(N // R,),
        in_specs=[_acc_spec(0), _acc_spec(1), _row_spec(H), _row_spec(16),
                  _row_spec(16), _full_spec((1, H)), _full_spec((H, H))],
        out_specs=_row_spec(H),
        out_shape=jax.ShapeDtypeStruct((N, H), jnp.float32),
    )(acc, acc, hs, deg0, deg1, b, w)


def _tc_head(acc, hs, deg0, deg1, b3, wl1, bl1, wl2p, bl2p):
    return pl.pallas_call(
        _head_body,
        grid=(N // R,),
        in_specs=[_acc_spec(0), _acc_spec(1), _row_spec(H), _row_spec(16),
                  _row_spec(16), _full_spec((1, H)), _full_spec((H, H)),
                  _full_spec((1, H)), _full_spec((H, H)), _full_spec((1, H))],
        out_specs=_row_spec(H),
        out_shape=jax.ShapeDtypeStruct((N, H), jnp.float32),
    )(acc, acc, hs, deg0, deg1, b3, wl1, bl1, wl2p, bl2p)


def kernel(x, edge_index, batch, W1, b1, W2, b2, W3, b3, Wl1, bl1, Wl2, bl2):
    del batch
    pad = jnp.arange(EPAD, dtype=jnp.int32)
    srcp = jnp.concatenate([edge_index[0], pad % N])
    dstp = jnp.concatenate([edge_index[1], N + pad % (NP - N)])
    dst_r = dstp.reshape(NW, NCHUNK, CS)
    idx_packed = jnp.stack([srcp.reshape(NW, NWIN, G, CS),
                            dstp.reshape(NW, NWIN, G, CS)], axis=2)
    ones128 = jnp.ones((CS, H), jnp.float32)
    zeros128 = jnp.zeros((CS, H), jnp.float32)

    deg = _sc_degree(dst_r, ones128, zeros128)
    deg0, deg1 = deg[0, :N, :16], deg[1, :N, :16]

    hs = _tc_first(x, W1, deg0, deg1)
    acc = _sc_accumulate(hs, idx_packed, zeros128)
    hs = _tc_mid(acc, hs, deg0, deg1, b1.reshape(1, H), W2)
    acc = _sc_accumulate(hs, idx_packed, zeros128)
    hs = _tc_mid(acc, hs, deg0, deg1, b2.reshape(1, H), W3)
    acc = _sc_accumulate(hs, idx_packed, zeros128)

    wl2p = jnp.pad(Wl2, ((0, 0), (0, H - C)))
    bl2p = jnp.pad(bl2, (0, H - C)).reshape(1, H)
    out = _tc_head(acc, hs, deg0, deg1, b3.reshape(1, H),
                   Wl1, bl1.reshape(1, H), wl2p, bl2p)
    return out[:, :C]


# speedup vs baseline: 2.8415x; 1.0176x over previous
import jax
import jax.numpy as jnp
from jax import lax
from jax.experimental import pallas as pl
from jax.experimental.pallas import tpu as pltpu
from jax.experimental.pallas import tpu_sc as plsc

N = 10000
E = 320000
H = 128
C = 40

NC = 2
NS = 16
NW = NC * NS
CS = 128
NCHUNK = 80
EPWP = NCHUNK * CS
EPAD = NW * EPWP - E
G = 4
NWIN = NCHUNK // G
NSW = NWIN // 2
NP = 10240
RPT = NP // NS


def _sc_mesh():
    return plsc.VectorSubcoreMesh(core_axis_name="c", subcore_axis_name="s")


def _zero_stripe(zeros_hbm, buf_v, sh, row0, sem):
    pltpu.sync_copy(zeros_hbm, buf_v)
    for i in range(RPT // CS):
        pltpu.async_copy(buf_v, sh.at[pl.ds(row0 + i * CS, CS)], sem)
    for _ in range(RPT // CS):
        pltpu.make_async_copy(buf_v, sh.at[pl.ds(row0, CS)], sem).wait()


def _writeback_stripe(sh, out_hbm, cid, row0, bufs, sems):
    npiece = RPT // CS
    for i in range(npiece):
        b = i % 2
        if i >= 2:
            pltpu.make_async_copy(bufs[b], out_hbm.at[cid, pl.ds(row0, CS)],
                                  sems[b]).wait()
        pltpu.sync_copy(sh.at[pl.ds(row0 + i * CS, CS)], bufs[b])
        pltpu.async_copy(bufs[b], out_hbm.at[cid, pl.ds(row0 + i * CS, CS)],
                         sems[b])
    for i in range(npiece - 2, npiece):
        b = i % 2
        pltpu.make_async_copy(bufs[b], out_hbm.at[cid, pl.ds(row0, CS)],
                              sems[b]).wait()


def _deg_body(dst_hbm, ones_hbm, zeros_hbm, out_hbm,
              deg_sh, dst_v, ones_v, zbuf, sem0, sem1):
    cid = lax.axis_index("c")
    sid = lax.axis_index("s")
    wid = cid * NS + sid
    row0 = sid * RPT
    _zero_stripe(zeros_hbm, zbuf, deg_sh, row0, sem0)
    pltpu.sync_copy(ones_hbm, ones_v)
    pltpu.sync_copy(dst_hbm.at[wid], dst_v)
    plsc.subcore_barrier()

    pltpu.async_copy(ones_v, deg_sh.at[dst_v.at[0]], sem0, add=True)

    def chunk(j, carry):
        pltpu.async_copy(ones_v, deg_sh.at[dst_v.at[j]], sem0, add=True)
        pltpu.make_async_copy(ones_v, deg_sh.at[dst_v.at[0]], sem0).wait()
        return carry

    lax.fori_loop(1, NCHUNK, chunk, None)
    pltpu.make_async_copy(ones_v, deg_sh.at[dst_v.at[0]], sem0).wait()
    plsc.subcore_barrier()
    _writeback_stripe(deg_sh, out_hbm, cid, row0, (ones_v, zbuf), (sem0, sem1))


def _sc_degree(dst_r, ones128, zeros128):
    f = pl.kernel(
        _deg_body,
        out_type=jax.ShapeDtypeStruct((NC, NP, H), jnp.float32),
        mesh=_sc_mesh(),
        scratch_types=[
            pltpu.VMEM_SHARED((NP, H), jnp.float32),
            pltpu.VMEM((NCHUNK, CS), jnp.int32),
            pltpu.VMEM((CS, H), jnp.float32),
            pltpu.VMEM((CS, H), jnp.float32),
            pltpu.SemaphoreType.DMA,
            pltpu.SemaphoreType.DMA,
        ],
    )
    return f(dst_r, ones128, zeros128)


def _acc_body(h_hbm, idx_hbm, zeros_hbm, out_hbm,
              acc_sh, slot0, slot1, rows0, rows1,
              semw0, semw1, semg0, semg1):
    cid = lax.axis_index("c")
    sid = lax.axis_index("s")
    wid = cid * NS + sid
    row0 = sid * RPT
    rows = (rows0, rows1)
    semg = (semg0, semg1)
    _zero_stripe(zeros_hbm, rows0, acc_sh, row0, semg0)
    plsc.subcore_barrier()

    pltpu.async_copy(idx_hbm.at[wid, 0], slot0, semw0).wait()
    pltpu.async_copy(idx_hbm.at[wid, 1], slot1, semw1)
    pltpu.async_copy(h_hbm.at[slot0.at[0, 0]], rows0, semg0)

    def chunk(slot, nslot, k, p):
        pltpu.make_async_copy(h_hbm.at[slot.at[0, k]], rows[p], semg[p]).wait()
        nk = 0 if k == G - 1 else k + 1
        nsl = nslot if k == G - 1 else slot
        if nsl is not None:
            pltpu.async_copy(h_hbm.at[nsl.at[0, nk]], rows[1 - p], semg[1 - p])
        pltpu.sync_copy(rows[p], acc_sh.at[slot.at[1, k]], add=True)

    def superwindow(w, carry):
        pltpu.make_async_copy(idx_hbm.at[wid, 0], slot1, semw1).wait()
        for k in range(G):
            chunk(slot0, slot1, k, k % 2)

        @pl.when(w < NSW - 1)
        def _fetch_even():
            pltpu.async_copy(idx_hbm.at[wid, 2 * w + 2], slot0, semw0)

        for k in range(G - 1):
            chunk(slot1, None, k, k % 2)

        @pl.when(w < NSW - 1)
        def _last_mid():
            pltpu.make_async_copy(idx_hbm.at[wid, 0], slot0, semw0).wait()
            chunk(slot1, slot0, G - 1, (G - 1) % 2)
            pltpu.async_copy(idx_hbm.at[wid, 2 * w + 3], slot1, semw1)

        @pl.when(w == NSW - 1)
        def _last_end():
            p = (G - 1) % 2
            pltpu.make_async_copy(h_hbm.at[slot1.at[0, G - 1]], rows[p],
                                  semg[p]).wait()
            pltpu.sync_copy(rows[p], acc_sh.at[slot1.at[1, G - 1]], add=True)

        return carry

    lax.fori_loop(0, NSW, superwindow, None)
    plsc.subcore_barrier()
    _writeback_stripe(acc_sh, out_hbm, cid, row0, rows, semg)


def _sc_accumulate(h_s, idx_packed, zeros128):
    f = pl.kernel(
        _acc_body,
        out_type=jax.ShapeDtypeStruct((NC, NP, H), jnp.float32),
        mesh=_sc_mesh(),
        scratch_types=[
            pltpu.VMEM_SHARED((NP, H), jnp.float32),
            pltpu.VMEM((2, G, CS), jnp.int32),
            pltpu.VMEM((2, G, CS), jnp.int32),
            pltpu.VMEM((CS, H), jnp.float32),
            pltpu.VMEM((CS, H), jnp.float32),
            pltpu.SemaphoreType.DMA,
            pltpu.SemaphoreType.DMA,
            pltpu.SemaphoreType.DMA,
            pltpu.SemaphoreType.DMA,
        ],
    )
    return f(h_s, idx_packed, zeros128)


R = 400


def _dinv(deg0, deg1):
    return lax.rsqrt(deg0[:, 0:1] + deg1[:, 0:1] + 1.0)


def _first_body(x_ref, w_ref, deg0_ref, deg1_ref, o_ref):
    dinv = _dinv(deg0_ref[...], deg1_ref[...])
    h = jnp.dot(x_ref[...], w_ref[...], preferred_element_type=jnp.float32)
    o_ref[...] = h * dinv


def _mid_body(a0_ref, a1_ref, hs_ref, deg0_ref, deg1_ref, b_ref, w_ref, o_ref):
    dinv = _dinv(deg0_ref[...], deg1_ref[...])
    z = a0_ref[0] + a1_ref[0] + hs_ref[...]
    z = jnp.maximum(z * dinv + b_ref[...], 0.0)
    o_ref[...] = jnp.dot(z, w_ref[...], preferred_element_type=jnp.float32) * dinv


def _head_body(a0_ref, a1_ref, hs_ref, deg0_ref, deg1_ref, b3_ref,
               wl1_ref, bl1_ref, wl2_ref, bl2_ref, o_ref):
    dinv = _dinv(deg0_ref[...], deg1_ref[...])
    z = a0_ref[0] + a1_ref[0] + hs_ref[...]
    z = jnp.maximum(z * dinv + b3_ref[...], 0.0)
    h4 = jnp.dot(z, wl1_ref[...], preferred_element_type=jnp.float32)
    h4 = jnp.maximum(h4 + bl1_ref[...], 0.0)
    logits = jnp.dot(h4, wl2_ref[...], preferred_element_type=jnp.float32)
    logits = logits + bl2_ref[...]
    col = lax.broadcasted_iota(jnp.int32, logits.shape, 1)
    logits = jnp.where(col < C, logits, -1e30)
    m = jnp.max(logits, axis=-1, keepdims=True)
    lse = jnp.log(jnp.sum(jnp.exp(logits - m), axis=-1, keepdims=True))
    o_ref[...] = logits - m - lse


def _row_spec(width):
    return pl.BlockSpec((R, width), lambda i: (i, 0))


def _acc_spec(core):
    return pl.BlockSpec((1, R, H), lambda i, core=core: (core, i, 0))


def _full_spec(shape):
    return pl.BlockSpec(shape, lambda i: (0,) * len(shape))


def _tc_first(x, w, deg0, deg1):
    return pl.pallas_call(
        _first_body,
        grid=(N // R,),
        in_specs=[_row_spec(H), _full_spec((H, H)), _row_spec(16), _row_spec(16)],
        out_specs=_row_spec(H),
        out_shape=jax.ShapeDtypeStruct((N, H), jnp.float32),
    )(x, w, deg0, deg1)


def _tc_mid(acc, hs, deg0, deg1, b, w):
    return pl.pallas_call(
        _mid_body,
        grid=(N // R,),
        in_specs=[_acc_spec(0), _acc_spec(1), _row_spec(H), _row_spec(16),
                  _row_spec(16), _full_spec((1, H)), _full_spec((H, H))],
        out_specs=_row_spec(H),
        out_shape=jax.ShapeDtypeStruct((N, H), jnp.float32),
    )(acc, acc, hs, deg0, deg1, b, w)


def _tc_head(acc, hs, deg0, deg1, b3, wl1, bl1, wl2p, bl2p):
    return pl.pallas_call(
        _head_body,
        grid=(N // R,),
        in_specs=[_acc_spec(0), _acc_spec(1), _row_spec(H), _row_spec(16),
                  _row_spec(16), _full_spec((1, H)), _full_spec((H, H)),
                  _full_spec((1, H)), _full_spec((H, H)), _full_spec((1, H))],
        out_specs=_row_spec(H),
        out_shape=jax.ShapeDtypeStruct((N, H), jnp.float32),
    )(acc, acc, hs, deg0, deg1, b3, wl1, bl1, wl2p, bl2p)


def kernel(x, edge_index, batch, W1, b1, W2, b2, W3, b3, Wl1, bl1, Wl2, bl2):
    del batch
    pad = jnp.arange(EPAD, dtype=jnp.int32)
    srcp = jnp.concatenate([edge_index[0], pad % N])
    dstp = jnp.concatenate([edge_index[1], N + pad % (NP - N)])
    dst_r = dstp.reshape(NW, NCHUNK, CS)
    idx_packed = jnp.stack([srcp.reshape(NW, NWIN, G, CS),
                            dstp.reshape(NW, NWIN, G, CS)], axis=2)
    ones128 = jnp.ones((CS, H), jnp.float32)
    zeros128 = jnp.zeros((CS, H), jnp.float32)

    deg = _sc_degree(dst_r, ones128, zeros128)
    deg0, deg1 = deg[0, :N, :16], deg[1, :N, :16]

    hs = _tc_first(x, W1, deg0, deg1)
    acc = _sc_accumulate(hs, idx_packed, zeros128)
    hs = _tc_mid(acc, hs, deg0, deg1, b1.reshape(1, H), W2)
    acc = _sc_accumulate(hs, idx_packed, zeros128)
    hs = _tc_mid(acc, hs, deg0, deg1, b2.reshape(1, H), W3)
    acc = _sc_accumulate(hs, idx_packed, zeros128)

    wl2p = jnp.pad(Wl2, ((0, 0), (0, H - C)))
    bl2p = jnp.pad(bl2, (0, H - C)).reshape(1, H)
    out = _tc_head(acc, hs, deg0, deg1, b3.reshape(1, H),
                   Wl1, bl1.reshape(1, H), wl2p, bl2p)
    return out[:, :C]
